# Initial kernel scaffold; baseline (speedup 1.0000x reference)
import jax
import jax.numpy as jnp
from jax.experimental import pallas as pl


def kernel(x, edge_index, batch, Wg1, bg1, gamma1, beta1, p1, Wg2, bg2, gamma2, beta2, p2, Wfc, bfc, Wfc1, bfc1):
    raise NotImplementedError("write your pallas kernel here")


# speedup vs baseline: 14.3498x; 14.3498x over previous
import functools

import jax
import jax.numpy as jnp
from jax import lax
from jax.experimental import pallas as pl
from jax.experimental.pallas import tpu as pltpu
from jax.experimental.pallas import tpu_sc as plsc

N = 10000
NPAD = 10240
E = 160000
EPAD = 163840
ER = EPAD // 128
NW = 32
RPW = ER // NW
D = 128
H = 1024
K1 = 5000
K2 = 2500
RB = 1024
NB = NPAD // RB
NEG_INF = float("-inf")


def _sc_edge_scalar(tbl, src2d, dst2d):
    mesh = plsc.VectorSubcoreMesh(core_axis_name="c", subcore_axis_name="s")

    @functools.partial(
        pl.kernel,
        mesh=mesh,
        out_type=jax.ShapeDtypeStruct((2, NPAD), jnp.float32),
        scratch_types=[
            pltpu.VMEM((RPW, 128), jnp.int32),
            pltpu.VMEM((RPW, 128), jnp.int32),
            pltpu.VMEM((128,), jnp.float32),
            pltpu.VMEM((640,), jnp.float32),
            pltpu.VMEM_SHARED((NPAD,), jnp.float32),
            pltpu.SemaphoreType.DMA,
        ],
    )
    def k(tbl_hbm, src_hbm, dst_hbm, out_hbm, src_v, dst_v, val_v, zb, shared, sem):
        c = lax.axis_index("c")
        s = lax.axis_index("s")
        wid = s * 2 + c
        for t in range(40):
            zb[pl.ds(t * 16, 16)] = jnp.zeros((16,), jnp.float32)
        pltpu.sync_copy(zb, shared.at[pl.ds(s * 640, 640)])
        base = wid * RPW
        pltpu.sync_copy(src_hbm.at[pl.ds(base, RPW)], src_v)
        pltpu.sync_copy(dst_hbm.at[pl.ds(base, RPW)], dst_v)
        plsc.subcore_barrier()
        for j in range(RPW):
            pltpu.async_copy(tbl_hbm.at[src_v.at[j]], val_v, sem).wait()
            pltpu.sync_copy(val_v, shared.at[dst_v.at[j]], add=True)
        plsc.subcore_barrier()
        pltpu.sync_copy(shared.at[pl.ds(s * 640, 640)],
                        out_hbm.at[c, pl.ds(s * 640, 640)])

    return k(tbl, src2d, dst2d)


def _sc_edge_rows(y, src2d, dst2d, nchunks):
    mesh = plsc.VectorSubcoreMesh(core_axis_name="c", subcore_axis_name="s")

    @functools.partial(
        pl.kernel,
        mesh=mesh,
        out_type=jax.ShapeDtypeStruct((2, nchunks, NPAD, 128), jnp.float32),
        scratch_types=[
            pltpu.VMEM((RPW, 128), jnp.int32),
            pltpu.VMEM((RPW, 128), jnp.int32),
            pltpu.VMEM((2, 128, 128), jnp.float32),
            pltpu.VMEM((16, 128), jnp.float32),
            pltpu.VMEM_SHARED((NPAD, 128), jnp.float32),
            pltpu.SemaphoreType.DMA,
            pltpu.SemaphoreType.DMA,
        ],
    )
    def k(y_hbm, src_hbm, dst_hbm, out_hbm, src_v, dst_v, rows_v, zb, shared, sem0, sem1):
        c = lax.axis_index("c")
        s = lax.axis_index("s")
        wid = s * 2 + c
        base = wid * RPW

        def zrow(r, _):
            for t in range(8):
                zb[r, pl.ds(t * 16, 16)] = jnp.zeros((16,), jnp.float32)
            return 0

        lax.fori_loop(0, 16, zrow, 0)
        pltpu.sync_copy(src_hbm.at[pl.ds(base, RPW)], src_v)
        pltpu.sync_copy(dst_hbm.at[pl.ds(base, RPW)], dst_v)
        sems = [sem0, sem1]
        for ch in range(nchunks):
            for t in range(40):
                pltpu.sync_copy(zb, shared.at[pl.ds(s * 640 + t * 16, 16)])
            plsc.subcore_barrier()
            cp0 = pltpu.async_copy(y_hbm.at[ch].at[src_v.at[0]], rows_v.at[0], sems[0])
            for j in range(RPW):
                nxt = None
                if j + 1 < RPW:
                    nxt = pltpu.async_copy(y_hbm.at[ch].at[src_v.at[j + 1]],
                                           rows_v.at[(j + 1) % 2], sems[(j + 1) % 2])
                if j == 0:
                    cp0.wait()
                else:
                    pass
                pltpu.sync_copy(rows_v.at[j % 2], shared.at[dst_v.at[j]], add=True)
                if nxt is not None:
                    nxt.wait()
            plsc.subcore_barrier()
            pltpu.sync_copy(shared.at[pl.ds(s * 640, 640)],
                            out_hbm.at[c, ch, pl.ds(s * 640, 640)])
            plsc.subcore_barrier()

    return k(y, src2d, dst2d)


def _pre(raw3, xin, w_col, chunks):
    def body(raw_ref, x_ref, *rest):
        if w_col is None:
            (y_ref, dis_ref) = rest
            wv = None
        else:
            (w_ref, y_ref, dis_ref) = rest
            wv = w_ref[...]
        deg = raw_ref[0] + raw_ref[1] + 1.0
        dis = lax.rsqrt(deg)
        dis_ref[...] = dis
        scale = dis if wv is None else dis * wv
        y_ref[0] = x_ref[...] * scale

    in_specs = [
        pl.BlockSpec((2, RB, 1), lambda i, c: (0, i, 0)),
        pl.BlockSpec((RB, 128), lambda i, c: (i, c)),
    ]
    ins = [raw3, xin]
    if w_col is not None:
        in_specs.append(pl.BlockSpec((RB, 1), lambda i, c: (i, 0)))
        ins.append(w_col)
    return pl.pallas_call(
        body,
        grid=(NB, chunks),
        in_specs=in_specs,
        out_specs=[pl.BlockSpec((1, RB, 128), lambda i, c: (c, i, 0)),
                   pl.BlockSpec((RB, 1), lambda i, c: (i, 0))],
        out_shape=[jax.ShapeDtypeStruct((chunks, NPAD, 128), jnp.float32),
                   jax.ShapeDtypeStruct((NPAD, 1), jnp.float32)],
    )(*ins)


def _mm_stats(yparts, dis_col, W, b, mrow, kchunks):
    Hout = W.shape[1]
    KC = kchunks

    def body(ya_ref, yb_ref, dis_ref, w_ref, b_ref, m_ref, h_ref, s1_ref, s2_ref):
        i = pl.program_id(0)
        kk = pl.program_id(1)
        ysum = ya_ref[0, 0] + ya_ref[1, 0] + yb_ref[0]
        lhs = ysum * dis_ref[...]
        part = jnp.dot(lhs, w_ref[0], preferred_element_type=jnp.float32)

        @pl.when(kk == 0)
        def _():
            h_ref[...] = part

        @pl.when(kk > 0)
        def _():
            h_ref[...] += part

        @pl.when(kk == KC - 1)
        def _():
            h = h_ref[...] + b_ref[...]
            h_ref[...] = h
            mr = m_ref[...]
            c1 = jnp.dot(mr, h, preferred_element_type=jnp.float32, precision=lax.Precision.HIGHEST)
            c2 = jnp.dot(mr, h * h, preferred_element_type=jnp.float32, precision=lax.Precision.HIGHEST)

            @pl.when(i == 0)
            def _():
                s1_ref[...] = c1
                s2_ref[...] = c2

            @pl.when(i > 0)
            def _():
                s1_ref[...] += c1
                s2_ref[...] += c2

    raw, ys = yparts
    return pl.pallas_call(
        body,
        grid=(NB, KC),
        in_specs=[
            pl.BlockSpec((2, 1, RB, 128), lambda i, k: (0, k, i, 0)),
            pl.BlockSpec((1, RB, 128), lambda i, k: (k, i, 0)),
            pl.BlockSpec((RB, 1), lambda i, k: (i, 0)),
            pl.BlockSpec((1, 128, Hout), lambda i, k: (k, 0, 0)),
            pl.BlockSpec((1, Hout), lambda i, k: (0, 0)),
            pl.BlockSpec((1, RB), lambda i, k: (0, i)),
        ],
        out_specs=[pl.BlockSpec((RB, Hout), lambda i, k: (i, 0)),
                   pl.BlockSpec((1, Hout), lambda i, k: (0, 0)),
                   pl.BlockSpec((1, Hout), lambda i, k: (0, 0))],
        out_shape=[jax.ShapeDtypeStruct((NPAD, Hout), jnp.float32),
                   jax.ShapeDtypeStruct((1, Hout), jnp.float32),
                   jax.ShapeDtypeStruct((1, Hout), jnp.float32)],
    )(raw, ys, dis_col, W.reshape(KC, 128, Hout), b.reshape(1, Hout), mrow)


def _norm_score(h, s1, s2, gamma, beta, p_col, m_col, cnt):
    Hh = h.shape[1]

    def body(h_ref, s1_ref, s2_ref, g_ref, bt_ref, p_ref, m_ref, hr_ref, sc_ref):
        mu = s1_ref[...] / cnt
        var = s2_ref[...] / cnt - mu * mu
        gs = g_ref[...] * lax.rsqrt(var + 1e-5)
        hr = jnp.maximum((h_ref[...] - mu) * gs + bt_ref[...], 0.0)
        hr_ref[...] = hr
        pn2 = jnp.sum(p_ref[...] * p_ref[...])
        sc = jnp.dot(hr, p_ref[...], preferred_element_type=jnp.float32, precision=lax.Precision.HIGHEST)
        sc = jnp.tanh(sc * lax.rsqrt(pn2))
        sc_ref[...] = jnp.where(m_ref[...] > 0, sc, NEG_INF)

    return pl.pallas_call(
        body,
        grid=(NB,),
        in_specs=[
            pl.BlockSpec((RB, Hh), lambda i: (i, 0)),
            pl.BlockSpec((1, Hh), lambda i: (0, 0)),
            pl.BlockSpec((1, Hh), lambda i: (0, 0)),
            pl.BlockSpec((1, Hh), lambda i: (0, 0)),
            pl.BlockSpec((1, Hh), lambda i: (0, 0)),
            pl.BlockSpec((Hh, 1), lambda i: (0, 0)),
            pl.BlockSpec((RB, 1), lambda i: (i, 0)),
        ],
        out_specs=[pl.BlockSpec((RB, Hh), lambda i: (i, 0)),
                   pl.BlockSpec((RB, 1), lambda i: (i, 0))],
        out_shape=[jax.ShapeDtypeStruct((NPAD, Hh), jnp.float32),
                   jax.ShapeDtypeStruct((NPAD, 1), jnp.float32)],
    )(h, s1, s2, gamma.reshape(1, Hh), beta.reshape(1, Hh), p_col, m_col)


def _topk(score2d, kk):
    def body(s_ref, w_ref, m_ref):
        s = s_ref[...]
        b = lax.bitcast_convert_type(s, jnp.int32)
        u = lax.bitcast_convert_type(s, jnp.uint32)
        ku = jnp.where(b < 0, ~u, u | jnp.uint32(0x80000000))

        def bit_step(i, t):
            cand = t | (jnp.uint32(1) << (31 - i).astype(jnp.uint32))
            cnt = jnp.sum(jnp.where(ku >= cand, 1, 0))
            return jnp.where(cnt >= kk, cand, t)

        t = lax.fori_loop(0, 32, bit_step, jnp.uint32(0))
        cnt_gt = jnp.sum(jnp.where(ku > t, 1, 0))
        need = kk - cnt_gt
        rid = (lax.broadcasted_iota(jnp.int32, (80, 128), 0) * 128
               + lax.broadcasted_iota(jnp.int32, (80, 128), 1))
        ties = ku == t

        def idx_step(i, m):
            cand = m | (jnp.int32(1) << (13 - i))
            g = jnp.sum(jnp.where(ties & (rid < cand), 1, 0))
            return jnp.where(g < need, cand, m)

        bstar = lax.fori_loop(0, 14, idx_step, jnp.int32(0)) + 1
        sel = (ku > t) | (ties & (rid < bstar))
        w_ref[...] = jnp.where(sel, s, 0.0)
        m_ref[...] = jnp.where(sel, 1.0, 0.0)

    return pl.pallas_call(
        body,
        out_shape=[jax.ShapeDtypeStruct((80, 128), jnp.float32),
                   jax.ShapeDtypeStruct((80, 128), jnp.float32)],
    )(score2d)


def _vecmat(vrow, mat, scale):
    Hh = mat.shape[1]

    def body(v_ref, m_ref, o_ref):
        i = pl.program_id(0)
        c = jnp.dot(v_ref[...], m_ref[...], preferred_element_type=jnp.float32, precision=lax.Precision.HIGHEST)

        @pl.when(i == 0)
        def _():
            o_ref[...] = c

        @pl.when(i > 0)
        def _():
            o_ref[...] += c

        @pl.when(i == NB - 1)
        def _():
            o_ref[...] = o_ref[...] * scale

    return pl.pallas_call(
        body,
        grid=(NB,),
        in_specs=[pl.BlockSpec((1, RB), lambda i: (0, i)),
                  pl.BlockSpec((RB, Hh), lambda i: (i, 0))],
        out_specs=pl.BlockSpec((1, Hh), lambda i: (0, 0)),
        out_shape=jax.ShapeDtypeStruct((1, Hh), jnp.float32),
    )(vrow, mat)


def _head(x1, x2, Wfc, bfc, Wfc1p, bfc1p):
    def body(a_ref, b_ref, w1_ref, b1_ref, w2_ref, b2_ref, o_ref):
        z = a_ref[...] + b_ref[...]
        z = jnp.dot(z, w1_ref[...], preferred_element_type=jnp.float32, precision=lax.Precision.HIGHEST) + b1_ref[...]
        z = jnp.maximum(z, 0.0)
        o_ref[...] = jnp.dot(z, w2_ref[...], preferred_element_type=jnp.float32, precision=lax.Precision.HIGHEST) + b2_ref[...]

    return pl.pallas_call(
        body,
        out_shape=jax.ShapeDtypeStruct((1, 128), jnp.float32),
    )(x1, x2, Wfc, bfc.reshape(1, 512), Wfc1p, bfc1p)


def kernel(x, edge_index, batch, Wg1, bg1, gamma1, beta1, p1, Wg2, bg2,
           gamma2, beta2, p2, Wfc, bfc, Wfc1, bfc1):
    f32 = jnp.float32
    pad_ids = jnp.int32(N) + (jnp.arange(EPAD - E, dtype=jnp.int32) % (NPAD - N))
    src = jnp.concatenate([edge_index[0].astype(jnp.int32), pad_ids]).reshape(ER, 128)
    dst = jnp.concatenate([edge_index[1].astype(jnp.int32), pad_ids]).reshape(ER, 128)

    xp = jnp.zeros((NPAD, D), f32).at[:N].set(x)
    ones_tbl = jnp.ones((NPAD,), f32)
    vrow = jnp.zeros((1, NPAD), f32).at[0, :N].set(1.0)
    vcol = vrow.reshape(NPAD, 1)

    raw1 = _sc_edge_scalar(ones_tbl, src, dst)
    y1, dis1 = _pre(raw1.reshape(2, NPAD, 1), xp, None, 1)
    agg1 = _sc_edge_rows(y1, src, dst, 1)
    h1, s1, s2 = _mm_stats([agg1, y1], dis1, Wg1, bg1, vrow, 1)
    h1r, sc1 = _norm_score(h1, s1, s2, gamma1, beta1, p1.reshape(H, 1), vcol, float(N))

    w1, m1 = _topk(sc1.reshape(80, 128), K1)
    w1col = w1.reshape(NPAD, 1)
    m1row = m1.reshape(1, NPAD)
    m1tbl = m1.reshape(NPAD)
    m1col = m1.reshape(NPAD, 1)
    x1 = _vecmat(w1.reshape(1, NPAD), h1r, 1.0 / K1)

    raw2 = _sc_edge_scalar(m1tbl, src, dst)
    y2t, dis2 = _pre(raw2.reshape(2, NPAD, 1), h1r, w1col, 8)
    agg2 = _sc_edge_rows(y2t, src, dst, 8)
    h2, t1, t2 = _mm_stats([agg2, y2t], dis2, Wg2, bg2, m1row, 8)
    h2r, sc2 = _norm_score(h2, t1, t2, gamma2, beta2, p2.reshape(H, 1), m1col, float(K1))

    w2, _m2 = _topk(sc2.reshape(80, 128), K2)
    x2 = _vecmat(w2.reshape(1, NPAD), h2r, 1.0 / K2)

    Wfc1p = jnp.zeros((512, 128), f32).at[:, :10].set(Wfc1)
    bfc1p = jnp.zeros((1, 128), f32).at[0, :10].set(bfc1)
    out = _head(x1, x2, Wfc, bfc, Wfc1p, bfc1p)
    return out[:, :10]
